# speedup vs baseline: 3.2759x; 3.2759x over previous
import jax
import jax.numpy as jnp
from jax.experimental import pallas as pl

D_MODEL = 1024
NUM_WAVES = 3
BLK = 1024


def _body(mask_ref, table_ref, out_ref):
    m = mask_ref[...]
    m0 = m[:, 0:1]
    m1 = m[:, 1:2]
    m2 = m[:, 2:3]
    l0 = jnp.logical_and(m0 >= m1, m0 >= m2)
    l1 = jnp.logical_and(jnp.logical_not(l0), m1 >= m2)
    t0 = table_ref[0:1, :]
    t1 = table_ref[1:2, :]
    t2 = table_ref[2:3, :]
    out_ref[...] = jnp.where(l0, t0, jnp.where(l1, t1, t2))


def kernel(wave_mask, wave_embedding):
    B, S, W = wave_mask.shape
    N = B * S
    mask2d = wave_mask.reshape(N, W)
    grid = (N // BLK,)
    out = pl.pallas_call(
        _body,
        grid=grid,
        in_specs=[
            pl.BlockSpec((BLK, W), lambda i: (i, 0)),
            pl.BlockSpec((NUM_WAVES, D_MODEL), lambda i: (0, 0)),
        ],
        out_specs=pl.BlockSpec((BLK, D_MODEL), lambda i: (i, 0)),
        out_shape=jax.ShapeDtypeStruct((N, D_MODEL), jnp.float32),
    )(mask2d, wave_embedding)
    return out.reshape(B, S, D_MODEL)
